# baseline (speedup 1.0000x reference)
import jax
import jax.numpy as jnp
from jax.experimental import pallas as pl
from jax.experimental.pallas import tpu as pltpu

_NA = 3
_NC = 80


def _decode_body(g_ref, s_ref, p_hbm, o_hbm, xbuf, tbuf, in_sem, out_sem):
    b = pl.program_id(0)
    nb = pl.num_programs(0)
    slot = jax.lax.rem(b, 2)
    nxt = 1 - slot
    ncell = g_ref.shape[1]
    nch = o_hbm.shape[0]

    half = ncell // 2

    def in_copy_h(bb, sl, h):
        sel = pl.ds(h * half, half)
        return pltpu.make_async_copy(
            p_hbm.at[sel, bb, :], xbuf.at[sl, sel, :], in_sem.at[sl, h]
        )

    def in_start(bb, sl):
        in_copy_h(bb, sl, 0).start()
        in_copy_h(bb, sl, 1).start()

    def in_wait(bb, sl):
        in_copy_h(bb, sl, 0).wait()
        in_copy_h(bb, sl, 1).wait()

    def out_copy(bb, sl):
        return pltpu.make_async_copy(
            tbuf.at[sl], o_hbm.at[:, bb, :], out_sem.at[sl]
        )

    @pl.when(b == 0)
    def _():
        in_start(b, slot)

    @pl.when(b + 1 < nb)
    def _():
        in_start(b + 1, nxt)

    in_wait(b, slot)

    @pl.when(b >= 2)
    def _():
        out_copy(b - 2, slot).wait()

    t = xbuf[slot].T

    stride = s_ref[_NA, 0]
    bias5 = jnp.pad(g_ref[...] * 1.0, ((0, 3), (0, 0)))
    row5 = jax.lax.broadcasted_iota(jnp.int32, (5, ncell), 0)
    for a in range(_NA):
        ta = t[nch * a : nch * (a + 1), :]
        w5 = ta[0:5, :]
        sg = jax.nn.sigmoid(w5)
        ex = jnp.exp(w5)
        f = jnp.where((row5 == 2) | (row5 == 3), ex, sg)
        scale5 = jnp.where(
            row5 < 2,
            stride,
            jnp.where(row5 == 2, s_ref[a, 0], jnp.where(row5 == 3, s_ref[a, 1], 1.0)),
        )
        tbuf[slot, :, pl.ds(ncell * a, ncell)] = ta
        tbuf[slot, 0:5, pl.ds(ncell * a, ncell)] = f * scale5 + bias5

    out_copy(b, slot).start()

    @pl.when(b == nb - 1)
    def _():
        out_copy(b, slot).wait()

        @pl.when(nb > 1)
        def _():
            out_copy(b - 1, nxt).wait()


def kernel(p, anchors, img_size):
    bs = p.shape[0]
    nG = p.shape[-1]
    nA = anchors.shape[0]
    nC = _NC
    ncell = nG * nG
    nch = nC + 5
    stride = jnp.float32(img_size / nG)

    pt = p.transpose(2, 3, 0, 1).reshape(ncell, bs, nA * nch)

    g = jnp.arange(ncell, dtype=jnp.float32)
    gx = jnp.remainder(g, nG) * stride
    gy = jnp.floor(g / nG) * stride
    g2 = jnp.stack([gx, gy], axis=0)

    scales = jnp.concatenate(
        [anchors.astype(jnp.float32), jnp.full((1, 2), stride, jnp.float32)], axis=0
    )

    out = pl.pallas_call(
        _decode_body,
        grid=(bs,),
        in_specs=[
            pl.BlockSpec((2, ncell), lambda b: (0, 0)),
            pl.BlockSpec(memory_space=pltpu.MemorySpace.SMEM),
            pl.BlockSpec(memory_space=pltpu.MemorySpace.HBM),
        ],
        out_specs=pl.BlockSpec(memory_space=pltpu.MemorySpace.HBM),
        out_shape=jax.ShapeDtypeStruct((nch, bs, nA * ncell), jnp.float32),
        scratch_shapes=[
            pltpu.VMEM((2, ncell, nA * nch), jnp.float32),
            pltpu.VMEM((2, nch, nA * ncell), jnp.float32),
            pltpu.SemaphoreType.DMA((2, 2)),
            pltpu.SemaphoreType.DMA((2,)),
        ],
    )(g2, scales, pt)

    return out.transpose(1, 2, 0)


# speedup vs baseline: 1.1108x; 1.1108x over previous
import jax
import jax.numpy as jnp
from jax.experimental import pallas as pl
from jax.experimental.pallas import tpu as pltpu

_NA = 3
_NC = 80


def _decode_body(g_ref, s_ref, p_hbm, o_hbm, xbuf, tbuf, in_sem, out_sem):
    b = pl.program_id(0)
    nb = pl.num_programs(0)
    slot = jax.lax.rem(b, 3)
    ncell = g_ref.shape[1]
    nch = o_hbm.shape[0]

    half = ncell // 2

    def in_copy_h(bb, sl, h):
        sel = pl.ds(h * half, half)
        return pltpu.make_async_copy(
            p_hbm.at[sel, bb, :], xbuf.at[sl, sel, :], in_sem.at[sl, h]
        )

    def in_start(bb, sl):
        in_copy_h(bb, sl, 0).start()
        in_copy_h(bb, sl, 1).start()

    def in_wait(bb, sl):
        in_copy_h(bb, sl, 0).wait()
        in_copy_h(bb, sl, 1).wait()

    def out_copy(bb, sl):
        return pltpu.make_async_copy(
            tbuf.at[sl], o_hbm.at[:, bb, :], out_sem.at[sl]
        )

    @pl.when(b == 0)
    def _():
        in_start(0, 0)

        @pl.when(nb > 1)
        def _():
            in_start(1, 1)

    @pl.when(b + 2 < nb)
    def _():
        in_start(b + 2, jax.lax.rem(b + 2, 3))

    in_wait(b, slot)

    @pl.when(b >= 3)
    def _():
        out_copy(b - 3, slot).wait()

    t = xbuf[slot].T

    stride = s_ref[_NA, 0]
    bias5 = jnp.pad(g_ref[...] * 1.0, ((0, 3), (0, 0)))
    row5 = jax.lax.broadcasted_iota(jnp.int32, (5, ncell), 0)
    for a in range(_NA):
        ta = t[nch * a : nch * (a + 1), :]
        w5 = ta[0:5, :]
        sg = jax.nn.sigmoid(w5)
        ex = jnp.exp(w5)
        f = jnp.where((row5 == 2) | (row5 == 3), ex, sg)
        scale5 = jnp.where(
            row5 < 2,
            stride,
            jnp.where(row5 == 2, s_ref[a, 0], jnp.where(row5 == 3, s_ref[a, 1], 1.0)),
        )
        tbuf[slot, :, pl.ds(ncell * a, ncell)] = ta
        tbuf[slot, 0:5, pl.ds(ncell * a, ncell)] = f * scale5 + bias5

    out_copy(b, slot).start()

    @pl.when(b == nb - 1)
    def _():
        out_copy(b, slot).wait()

        @pl.when(nb > 1)
        def _():
            out_copy(b - 1, jax.lax.rem(b + 2, 3)).wait()

        @pl.when(nb > 2)
        def _():
            out_copy(b - 2, jax.lax.rem(b + 1, 3)).wait()


def kernel(p, anchors, img_size):
    bs = p.shape[0]
    nG = p.shape[-1]
    nA = anchors.shape[0]
    nC = _NC
    ncell = nG * nG
    nch = nC + 5
    stride = jnp.float32(img_size / nG)

    pt = p.transpose(2, 3, 0, 1).reshape(ncell, bs, nA * nch)

    g = jnp.arange(ncell, dtype=jnp.float32)
    gx = jnp.remainder(g, nG) * stride
    gy = jnp.floor(g / nG) * stride
    g2 = jnp.stack([gx, gy], axis=0)

    scales = jnp.concatenate(
        [anchors.astype(jnp.float32), jnp.full((1, 2), stride, jnp.float32)], axis=0
    )

    out = pl.pallas_call(
        _decode_body,
        grid=(bs,),
        in_specs=[
            pl.BlockSpec((2, ncell), lambda b: (0, 0)),
            pl.BlockSpec(memory_space=pltpu.MemorySpace.SMEM),
            pl.BlockSpec(memory_space=pltpu.MemorySpace.HBM),
        ],
        out_specs=pl.BlockSpec(memory_space=pltpu.MemorySpace.HBM),
        out_shape=jax.ShapeDtypeStruct((nch, bs, nA * ncell), jnp.float32),
        scratch_shapes=[
            pltpu.VMEM((3, ncell, nA * nch), jnp.float32),
            pltpu.VMEM((3, nch, nA * ncell), jnp.float32),
            pltpu.SemaphoreType.DMA((3, 2)),
            pltpu.SemaphoreType.DMA((3,)),
        ],
    )(g2, scales, pt)

    return out.transpose(1, 2, 0)
